# baseline (speedup 1.0000x reference)
import dataclasses
import functools

import jax
import jax.numpy as jnp
from jax import lax
from jax.experimental import pallas as pl
from jax.experimental.pallas import tpu as pltpu
from jax.experimental.pallas import tpu_sc as plsc

N = 10000
D = 128
H1 = 64
H2 = 32

NPAD = 10240
BLK = 256
NCORE = 2
NSUB = 16
LANE = 16
NW = NCORE * NSUB
CH = 64
ROWS_PER_TILE = NPAD // NSUB

_NEG = -1e38


def _leaky(v):
    return jnp.maximum(v, 0.0) + 0.2 * jnp.minimum(v, 0.0)


def _pre_body(x_ref, wt_ref, b_ref, a8_ref, h_ref, asad_ref):
    i = pl.program_id(0)
    h = jnp.dot(x_ref[...], wt_ref[...], preferred_element_type=jnp.float32)
    h = h + b_ref[...]
    h_ref[...] = h
    asad = lax.dot_general(a8_ref[...], h, (((1,), (1,)), ((), ())),
                           preferred_element_type=jnp.float32)
    colid = i * BLK + lax.broadcasted_iota(jnp.int32, (8, BLK), 1)
    rowid = lax.broadcasted_iota(jnp.int32, (8, BLK), 0)
    asad_ref[...] = jnp.where((rowid == 0) & (colid >= N), _NEG, asad)


def _pre_call(xp, wt, b, a8):
    din, hdim = wt.shape
    nblk = NPAD // BLK
    return pl.pallas_call(
        _pre_body,
        grid=(nblk,),
        in_specs=[
            pl.BlockSpec((BLK, din), lambda i: (i, 0)),
            pl.BlockSpec((din, hdim), lambda i: (0, 0)),
            pl.BlockSpec((1, hdim), lambda i: (0, 0)),
            pl.BlockSpec((8, hdim), lambda i: (0, 0)),
        ],
        out_specs=[
            pl.BlockSpec((BLK, hdim), lambda i: (i, 0)),
            pl.BlockSpec((8, BLK), lambda i: (0, i)),
        ],
        out_shape=[
            jax.ShapeDtypeStruct((NPAD, hdim), jnp.float32),
            jax.ShapeDtypeStruct((8, NPAD), jnp.float32),
        ],
    )(xp, wt, b, a8)


def _mid_body(hdim, p0_ref, p1_ref, h_ref, atop_ref, abot_ref, w2t_ref,
              b2_ref, a82_ref, g2_ref, asad2_ref):
    i = pl.program_id(0)
    h = h_ref[...]
    v = (jnp.dot(h, atop_ref[...], preferred_element_type=jnp.float32)
         + jnp.dot(h, abot_ref[...], preferred_element_type=jnp.float32))
    wself = jnp.exp(_leaky(v))
    p = p0_ref[...] + p1_ref[...]
    num = p[:, :hdim] + wself * h
    den = p[:, hdim:hdim + 1] + wself
    out1 = num / jnp.where(den > 0, den, 1.0)
    h2 = jnp.dot(out1, w2t_ref[...], preferred_element_type=jnp.float32)
    h2 = h2 + b2_ref[...]
    g2_ref[...] = h2
    asad2 = lax.dot_general(a82_ref[...], h2, (((1,), (1,)), ((), ())),
                            preferred_element_type=jnp.float32)
    colid = i * BLK + lax.broadcasted_iota(jnp.int32, (8, BLK), 1)
    rowid = lax.broadcasted_iota(jnp.int32, (8, BLK), 0)
    asad2_ref[...] = jnp.where((rowid == 0) & (colid >= N), _NEG, asad2)


def _mid_call(p0, p1, h, atop, abot, w2t, b2, a82):
    hdim = h.shape[1]
    h2dim = w2t.shape[1]
    ws = p0.shape[1]
    nblk = NPAD // BLK
    return pl.pallas_call(
        functools.partial(_mid_body, hdim),
        grid=(nblk,),
        in_specs=[
            pl.BlockSpec((BLK, ws), lambda i: (i, 0)),
            pl.BlockSpec((BLK, ws), lambda i: (i, 0)),
            pl.BlockSpec((BLK, hdim), lambda i: (i, 0)),
            pl.BlockSpec((hdim, 1), lambda i: (0, 0)),
            pl.BlockSpec((hdim, 1), lambda i: (0, 0)),
            pl.BlockSpec((hdim, h2dim), lambda i: (0, 0)),
            pl.BlockSpec((1, h2dim), lambda i: (0, 0)),
            pl.BlockSpec((8, h2dim), lambda i: (0, 0)),
        ],
        out_specs=[
            pl.BlockSpec((BLK, h2dim), lambda i: (i, 0)),
            pl.BlockSpec((8, BLK), lambda i: (0, i)),
        ],
        out_shape=[
            jax.ShapeDtypeStruct((NPAD, h2dim), jnp.float32),
            jax.ShapeDtypeStruct((8, NPAD), jnp.float32),
        ],
    )(p0, p1, h, atop, abot, w2t, b2, a82)


def _post_body(hdim, q0_ref, q1_ref, h_ref, atop_ref, abot_ref, out_ref):
    h = h_ref[...]
    v = (jnp.dot(h, atop_ref[...], preferred_element_type=jnp.float32)
         + jnp.dot(h, abot_ref[...], preferred_element_type=jnp.float32))
    wself = jnp.exp(_leaky(v))
    q = q0_ref[...] + q1_ref[...]
    num = q[:, :hdim] + wself * h
    den = q[:, hdim:hdim + 1] + wself
    o = num / jnp.where(den > 0, den, 1.0)
    m = jnp.max(o, axis=1, keepdims=True)
    out_ref[...] = o - m - jnp.log(jnp.sum(jnp.exp(o - m), axis=1,
                                           keepdims=True))


def _post_call(q0, q1, h, atop, abot):
    hdim = h.shape[1]
    ws = q0.shape[1]
    nblk = NPAD // BLK
    return pl.pallas_call(
        functools.partial(_post_body, hdim),
        grid=(nblk,),
        in_specs=[
            pl.BlockSpec((BLK, ws), lambda i: (i, 0)),
            pl.BlockSpec((BLK, ws), lambda i: (i, 0)),
            pl.BlockSpec((BLK, hdim), lambda i: (i, 0)),
            pl.BlockSpec((hdim, 1), lambda i: (0, 0)),
            pl.BlockSpec((hdim, 1), lambda i: (0, 0)),
        ],
        out_specs=pl.BlockSpec((BLK, hdim), lambda i: (i, 0)),
        out_shape=jax.ShapeDtypeStruct((NPAD, hdim), jnp.float32),
    )(q0, q1, h, atop, abot)


def _make_sc_edge_kernel(hdim, nchunk):
    ws = hdim + LANE
    grp = hdim // LANE
    nbuf = 4
    mesh = plsc.VectorSubcoreMesh(core_axis_name="c", subcore_axis_name="s")
    cp = pltpu.CompilerParams(use_tc_tiling_on_sc=False)
    if "needs_layout_passes" in pltpu.CompilerParams.__dataclass_fields__:
        cp = dataclasses.replace(cp, needs_layout_passes=False)

    @functools.partial(
        pl.kernel,
        out_type=jax.ShapeDtypeStruct((NCORE, NPAD, ws), jnp.float32),
        mesh=mesh,
        compiler_params=cp,
        scratch_types=(
            [
                pltpu.VMEM((nchunk + nbuf, CH), jnp.int32),
                pltpu.VMEM((nchunk, CH), jnp.int32),
                pltpu.VMEM((NPAD,), jnp.float32),
                pltpu.VMEM((NPAD,), jnp.float32),
            ]
            + [pltpu.VMEM((CH, hdim), jnp.bfloat16)] * nbuf
            + [pltpu.VMEM((CH, ws), jnp.float32)] * nbuf
            + [pltpu.VMEM_SHARED((NPAD, ws), jnp.float32)]
            + [pltpu.SemaphoreType.DMA] * (2 * nbuf)
        ),
    )
    def sc_edges(g_hbm, asad_hbm, row_hbm, col_hbm, out_hbm,
                 rows_i, cols_i, as_t, ad_t, *rest):
        gbufs = rest[:nbuf]
        sbufs = rest[nbuf:2 * nbuf]
        acc = rest[2 * nbuf]
        gsems = rest[2 * nbuf + 1:3 * nbuf + 1]
        ssems = rest[3 * nbuf + 1:4 * nbuf + 1]
        cid = lax.axis_index("c")
        sid = lax.axis_index("s")
        wid = cid * NSUB + sid
        pltpu.sync_copy(asad_hbm.at[0], as_t)
        pltpu.sync_copy(asad_hbm.at[1], ad_t)
        pltpu.sync_copy(row_hbm.at[wid], rows_i.at[pl.ds(0, nchunk)])
        pltpu.sync_copy(col_hbm.at[wid], cols_i)

        zero16 = jnp.zeros((LANE,), jnp.float32)
        zero16i = jnp.zeros((LANE,), jnp.int32)
        for k in range(nbuf):
            for g in range(CH // LANE):
                rows_i[nchunk + k, pl.ds(g * LANE, LANE)] = zero16i

        @pl.loop(0, CH)
        def _zero_srows(e):
            for g in range(ws // LANE):
                for sbuf in sbufs:
                    sbuf[e, pl.ds(g * LANE, LANE)] = zero16

        base = sid * ROWS_PER_TILE
        for k in range(ROWS_PER_TILE // CH):
            pltpu.sync_copy(sbufs[0], acc.at[pl.ds(base + k * CH, CH)])
        plsc.subcore_barrier()

        iota16 = lax.iota(jnp.int32, LANE)
        colw = jnp.full((LANE,), hdim, jnp.int32)
        bufs = tuple(zip(gbufs, sbufs, gsems, ssems))

        for par, (gbuf, sbuf, gsem, ssem) in enumerate(bufs):
            pltpu.async_copy(g_hbm.at[rows_i.at[par]], gbuf, gsem)
            pltpu.async_copy(sbuf, acc.at[cols_i.at[0]], ssem, add=True)

        @pl.loop(0, nchunk // nbuf)
        def _pair(jj):
            for par, (gbuf, sbuf, gsem, ssem) in enumerate(bufs):
                j = jj * nbuf + par
                rsl = rows_i.at[j]
                csl = cols_i.at[j]
                pltpu.make_async_copy(sbuf, acc.at[csl], ssem).wait()
                pltpu.make_async_copy(g_hbm.at[rsl], gbuf, gsem).wait()

                @pl.loop(0, CH // LANE)
                def _group(g):
                    r16 = rsl[pl.ds(g * LANE, LANE)]
                    c16 = csl[pl.ds(g * LANE, LANE)]
                    v = (plsc.load_gather(as_t, [r16])
                         + plsc.load_gather(ad_t, [c16]))
                    w16 = jnp.exp(_leaky(v))
                    plsc.store_scatter(sbuf, [g * LANE + iota16, colw], w16)
                    for l in range(LANE):
                        wv = jnp.full((LANE,), w16[l])
                        e = g * LANE + l
                        for gg in range(hdim // 32):
                            ab = gbuf[e, pl.ds(gg * 32, 32)]
                            va, vb = plsc.unpack(
                                ab, format=plsc.PackFormat.INTERLEAVED,
                                preferred_element_type=jnp.float32)
                            sbuf[e, pl.ds(gg * 32, LANE)] = va * wv
                            sbuf[e, pl.ds(gg * 32 + LANE, LANE)] = vb * wv

                pltpu.async_copy(sbuf, acc.at[csl], ssem, add=True)
                pltpu.async_copy(g_hbm.at[rows_i.at[j + nbuf]], gbuf, gsem)

        for par, (gbuf, sbuf, gsem, ssem) in enumerate(bufs):
            pltpu.make_async_copy(g_hbm.at[rows_i.at[nchunk]], gbuf,
                                  gsem).wait()
            pltpu.make_async_copy(sbuf, acc.at[cols_i.at[0]], ssem).wait()

        plsc.subcore_barrier()
        for k in range(ROWS_PER_TILE // CH):
            pltpu.sync_copy(acc.at[pl.ds(base + k * CH, CH)],
                            out_hbm.at[cid, pl.ds(base + k * CH, CH)])

    return sc_edges


def _bf16_interleave(h):
    n, hd = h.shape
    hp = h.reshape(n, hd // 32, 2, 16).transpose(0, 1, 3, 2).reshape(n, hd)
    return hp.astype(jnp.bfloat16)


def _pack_a8(avec, hdim):
    atop = avec[:hdim, 0]
    abot = avec[hdim:, 0]
    return jnp.concatenate(
        [atop[None, :], abot[None, :], jnp.zeros((6, hdim), jnp.float32)], 0)


@jax.jit
def kernel(x, edge_index, W1, b1, a1, W2, b2, a2):
    xp = jnp.pad(x, ((0, NPAD - N), (0, 0)))
    row = edge_index[0].astype(jnp.int32)
    col = edge_index[1].astype(jnp.int32)
    e_total = row.shape[0]
    nchunk = -(-e_total // (NW * CH))
    nchunk = -(-nchunk // 4) * 4
    epad = NW * CH * nchunk
    padidx = N + (jnp.arange(epad - e_total, dtype=jnp.int32) % (NPAD - N))
    rowp = jnp.concatenate([row, padidx]).reshape(NW, nchunk, CH)
    colp = jnp.concatenate([col, padidx]).reshape(NW, nchunk, CH)

    atop1, abot1 = a1[:H1], a1[H1:]
    atop2, abot2 = a2[:H2], a2[H2:]

    h1, asad1 = _pre_call(xp, W1.T, b1.reshape(1, H1), _pack_a8(a1, H1))
    p = _make_sc_edge_kernel(H1, nchunk)(_bf16_interleave(h1), asad1,
                                         rowp, colp)
    h2, asad2 = _mid_call(p[0], p[1], h1, atop1, abot1, W2.T,
                          b2.reshape(1, H2), _pack_a8(a2, H2))
    q = _make_sc_edge_kernel(H2, nchunk)(_bf16_interleave(h2), asad2,
                                         rowp, colp)
    out = _post_call(q[0], q[1], h2, atop2, abot2)
    return out[:N]


# speedup vs baseline: 1.0597x; 1.0597x over previous
import dataclasses
import functools

import jax
import jax.numpy as jnp
from jax import lax
from jax.experimental import pallas as pl
from jax.experimental.pallas import tpu as pltpu
from jax.experimental.pallas import tpu_sc as plsc

N = 10000
D = 128
H1 = 64
H2 = 32

NPAD = 10240
BLK = 512
NCORE = 2
NSUB = 16
LANE = 16
NW = NCORE * NSUB
CH = 96
ROWS_PER_TILE = NPAD // NSUB

_NEG = -1e38


def _leaky(v):
    return jnp.maximum(v, 0.0) + 0.2 * jnp.minimum(v, 0.0)


def _pre_body(x_ref, wt_ref, b_ref, a8_ref, h_ref, asad_ref):
    i = pl.program_id(0)
    h = jnp.dot(x_ref[...], wt_ref[...], preferred_element_type=jnp.float32)
    h = h + b_ref[...]
    h_ref[...] = h
    asad = lax.dot_general(a8_ref[...], h, (((1,), (1,)), ((), ())),
                           preferred_element_type=jnp.float32)
    colid = i * BLK + lax.broadcasted_iota(jnp.int32, (8, BLK), 1)
    rowid = lax.broadcasted_iota(jnp.int32, (8, BLK), 0)
    asad_ref[...] = jnp.where((rowid == 0) & (colid >= N), _NEG, asad)


def _pre_call(xp, wt, b, a8):
    din, hdim = wt.shape
    nblk = NPAD // BLK
    return pl.pallas_call(
        _pre_body,
        grid=(nblk,),
        in_specs=[
            pl.BlockSpec((BLK, din), lambda i: (i, 0)),
            pl.BlockSpec((din, hdim), lambda i: (0, 0)),
            pl.BlockSpec((1, hdim), lambda i: (0, 0)),
            pl.BlockSpec((8, hdim), lambda i: (0, 0)),
        ],
        out_specs=[
            pl.BlockSpec((BLK, hdim), lambda i: (i, 0)),
            pl.BlockSpec((8, BLK), lambda i: (0, i)),
        ],
        out_shape=[
            jax.ShapeDtypeStruct((NPAD, hdim), jnp.float32),
            jax.ShapeDtypeStruct((8, NPAD), jnp.float32),
        ],
    )(xp, wt, b, a8)


def _mid_body(hdim, p0_ref, p1_ref, h_ref, atop_ref, abot_ref, w2t_ref,
              b2_ref, a82_ref, g2_ref, asad2_ref):
    i = pl.program_id(0)
    h = h_ref[...]
    v = (jnp.dot(h, atop_ref[...], preferred_element_type=jnp.float32)
         + jnp.dot(h, abot_ref[...], preferred_element_type=jnp.float32))
    wself = jnp.exp(_leaky(v))
    p = p0_ref[...] + p1_ref[...]
    num = p[:, :hdim] + wself * h
    den = p[:, hdim:hdim + 1] + wself
    out1 = num / jnp.where(den > 0, den, 1.0)
    h2 = jnp.dot(out1, w2t_ref[...], preferred_element_type=jnp.float32)
    h2 = h2 + b2_ref[...]
    g2_ref[...] = h2
    asad2 = lax.dot_general(a82_ref[...], h2, (((1,), (1,)), ((), ())),
                            preferred_element_type=jnp.float32)
    colid = i * BLK + lax.broadcasted_iota(jnp.int32, (8, BLK), 1)
    rowid = lax.broadcasted_iota(jnp.int32, (8, BLK), 0)
    asad2_ref[...] = jnp.where((rowid == 0) & (colid >= N), _NEG, asad2)


def _mid_call(p0, p1, h, atop, abot, w2t, b2, a82):
    hdim = h.shape[1]
    h2dim = w2t.shape[1]
    ws = p0.shape[1]
    nblk = NPAD // BLK
    return pl.pallas_call(
        functools.partial(_mid_body, hdim),
        grid=(nblk,),
        in_specs=[
            pl.BlockSpec((BLK, ws), lambda i: (i, 0)),
            pl.BlockSpec((BLK, ws), lambda i: (i, 0)),
            pl.BlockSpec((BLK, hdim), lambda i: (i, 0)),
            pl.BlockSpec((hdim, 1), lambda i: (0, 0)),
            pl.BlockSpec((hdim, 1), lambda i: (0, 0)),
            pl.BlockSpec((hdim, h2dim), lambda i: (0, 0)),
            pl.BlockSpec((1, h2dim), lambda i: (0, 0)),
            pl.BlockSpec((8, h2dim), lambda i: (0, 0)),
        ],
        out_specs=[
            pl.BlockSpec((BLK, h2dim), lambda i: (i, 0)),
            pl.BlockSpec((8, BLK), lambda i: (0, i)),
        ],
        out_shape=[
            jax.ShapeDtypeStruct((NPAD, h2dim), jnp.float32),
            jax.ShapeDtypeStruct((8, NPAD), jnp.float32),
        ],
    )(p0, p1, h, atop, abot, w2t, b2, a82)


def _post_body(hdim, q0_ref, q1_ref, h_ref, atop_ref, abot_ref, out_ref):
    h = h_ref[...]
    v = (jnp.dot(h, atop_ref[...], preferred_element_type=jnp.float32)
         + jnp.dot(h, abot_ref[...], preferred_element_type=jnp.float32))
    wself = jnp.exp(_leaky(v))
    q = q0_ref[...] + q1_ref[...]
    num = q[:, :hdim] + wself * h
    den = q[:, hdim:hdim + 1] + wself
    o = num / jnp.where(den > 0, den, 1.0)
    m = jnp.max(o, axis=1, keepdims=True)
    out_ref[...] = o - m - jnp.log(jnp.sum(jnp.exp(o - m), axis=1,
                                           keepdims=True))


def _post_call(q0, q1, h, atop, abot):
    hdim = h.shape[1]
    ws = q0.shape[1]
    nblk = NPAD // BLK
    return pl.pallas_call(
        functools.partial(_post_body, hdim),
        grid=(nblk,),
        in_specs=[
            pl.BlockSpec((BLK, ws), lambda i: (i, 0)),
            pl.BlockSpec((BLK, ws), lambda i: (i, 0)),
            pl.BlockSpec((BLK, hdim), lambda i: (i, 0)),
            pl.BlockSpec((hdim, 1), lambda i: (0, 0)),
            pl.BlockSpec((hdim, 1), lambda i: (0, 0)),
        ],
        out_specs=pl.BlockSpec((BLK, hdim), lambda i: (i, 0)),
        out_shape=jax.ShapeDtypeStruct((NPAD, hdim), jnp.float32),
    )(q0, q1, h, atop, abot)


def _make_sc_edge_kernel(hdim, nchunk):
    ws = hdim + LANE
    grp = hdim // LANE
    nbuf = 3
    mesh = plsc.VectorSubcoreMesh(core_axis_name="c", subcore_axis_name="s")
    cp = pltpu.CompilerParams(use_tc_tiling_on_sc=False)
    if "needs_layout_passes" in pltpu.CompilerParams.__dataclass_fields__:
        cp = dataclasses.replace(cp, needs_layout_passes=False)

    @functools.partial(
        pl.kernel,
        out_type=jax.ShapeDtypeStruct((NCORE, NPAD, ws), jnp.float32),
        mesh=mesh,
        compiler_params=cp,
        scratch_types=(
            [
                pltpu.VMEM((nchunk + nbuf, CH), jnp.int32),
                pltpu.VMEM((nchunk, CH), jnp.int32),
                pltpu.VMEM((NPAD,), jnp.float32),
                pltpu.VMEM((NPAD,), jnp.float32),
            ]
            + [pltpu.VMEM((CH, hdim), jnp.bfloat16)] * nbuf
            + [pltpu.VMEM((CH, ws), jnp.float32)] * nbuf
            + [pltpu.VMEM_SHARED((NPAD, ws), jnp.float32)]
            + [pltpu.SemaphoreType.DMA] * (2 * nbuf)
        ),
    )
    def sc_edges(g_hbm, asad_hbm, row_hbm, col_hbm, out_hbm,
                 rows_i, cols_i, as_t, ad_t, *rest):
        gbufs = rest[:nbuf]
        sbufs = rest[nbuf:2 * nbuf]
        acc = rest[2 * nbuf]
        gsems = rest[2 * nbuf + 1:3 * nbuf + 1]
        ssems = rest[3 * nbuf + 1:4 * nbuf + 1]
        cid = lax.axis_index("c")
        sid = lax.axis_index("s")
        wid = cid * NSUB + sid
        pltpu.sync_copy(asad_hbm.at[0], as_t)
        pltpu.sync_copy(asad_hbm.at[1], ad_t)
        pltpu.sync_copy(row_hbm.at[wid], rows_i.at[pl.ds(0, nchunk)])
        pltpu.sync_copy(col_hbm.at[wid], cols_i)

        zero16 = jnp.zeros((LANE,), jnp.float32)
        zero16i = jnp.zeros((LANE,), jnp.int32)
        for k in range(nbuf):
            for g in range(CH // LANE):
                rows_i[nchunk + k, pl.ds(g * LANE, LANE)] = zero16i

        @pl.loop(0, CH)
        def _zero_srows(e):
            for g in range(ws // LANE):
                for sbuf in sbufs:
                    sbuf[e, pl.ds(g * LANE, LANE)] = zero16

        base = sid * ROWS_PER_TILE
        stripes = [(off, min(CH, ROWS_PER_TILE - off))
                   for off in range(0, ROWS_PER_TILE, CH)]
        for off, sz in stripes:
            pltpu.sync_copy(sbufs[0].at[pl.ds(0, sz)],
                            acc.at[pl.ds(base + off, sz)])
        plsc.subcore_barrier()

        iota16 = lax.iota(jnp.int32, LANE)
        colw = jnp.full((LANE,), hdim, jnp.int32)
        bufs = tuple(zip(gbufs, sbufs, gsems, ssems))

        for par, (gbuf, sbuf, gsem, ssem) in enumerate(bufs):
            pltpu.async_copy(g_hbm.at[rows_i.at[par]], gbuf, gsem)
            pltpu.async_copy(sbuf, acc.at[cols_i.at[0]], ssem, add=True)

        @pl.loop(0, nchunk // nbuf)
        def _pair(jj):
            for par, (gbuf, sbuf, gsem, ssem) in enumerate(bufs):
                j = jj * nbuf + par
                rsl = rows_i.at[j]
                csl = cols_i.at[j]
                pltpu.make_async_copy(sbuf, acc.at[csl], ssem).wait()
                pltpu.make_async_copy(g_hbm.at[rsl], gbuf, gsem).wait()

                @pl.loop(0, CH // LANE)
                def _group(g):
                    r16 = rsl[pl.ds(g * LANE, LANE)]
                    c16 = csl[pl.ds(g * LANE, LANE)]
                    v = (plsc.load_gather(as_t, [r16])
                         + plsc.load_gather(ad_t, [c16]))
                    w16 = jnp.exp(_leaky(v))
                    plsc.store_scatter(sbuf, [g * LANE + iota16, colw], w16)
                    for l in range(LANE):
                        wv = jnp.full((LANE,), w16[l])
                        e = g * LANE + l
                        for gg in range(hdim // 32):
                            ab = gbuf[e, pl.ds(gg * 32, 32)]
                            va, vb = plsc.unpack(
                                ab, format=plsc.PackFormat.INTERLEAVED,
                                preferred_element_type=jnp.float32)
                            sbuf[e, pl.ds(gg * 32, LANE)] = va * wv
                            sbuf[e, pl.ds(gg * 32 + LANE, LANE)] = vb * wv

                pltpu.async_copy(sbuf, acc.at[csl], ssem, add=True)
                pltpu.async_copy(g_hbm.at[rows_i.at[j + nbuf]], gbuf, gsem)

        for par, (gbuf, sbuf, gsem, ssem) in enumerate(bufs):
            pltpu.make_async_copy(g_hbm.at[rows_i.at[nchunk]], gbuf,
                                  gsem).wait()
            pltpu.make_async_copy(sbuf, acc.at[cols_i.at[0]], ssem).wait()

        plsc.subcore_barrier()
        for off, sz in stripes:
            pltpu.sync_copy(acc.at[pl.ds(base + off, sz)],
                            out_hbm.at[cid, pl.ds(base + off, sz)])

    return sc_edges


def _bf16_interleave(h):
    n, hd = h.shape
    hp = h.reshape(n, hd // 32, 2, 16).transpose(0, 1, 3, 2).reshape(n, hd)
    return hp.astype(jnp.bfloat16)


def _pack_a8(avec, hdim):
    atop = avec[:hdim, 0]
    abot = avec[hdim:, 0]
    return jnp.concatenate(
        [atop[None, :], abot[None, :], jnp.zeros((6, hdim), jnp.float32)], 0)


@jax.jit
def kernel(x, edge_index, W1, b1, a1, W2, b2, a2):
    xp = jnp.pad(x, ((0, NPAD - N), (0, 0)))
    row = edge_index[0].astype(jnp.int32)
    col = edge_index[1].astype(jnp.int32)
    e_total = row.shape[0]
    nchunk = -(-e_total // (NW * CH))
    nchunk = -(-nchunk // 3) * 3
    epad = NW * CH * nchunk
    padidx = N + (jnp.arange(epad - e_total, dtype=jnp.int32) % (NPAD - N))
    rowp = jnp.concatenate([row, padidx]).reshape(NW, nchunk, CH)
    colp = jnp.concatenate([col, padidx]).reshape(NW, nchunk, CH)

    atop1, abot1 = a1[:H1], a1[H1:]
    atop2, abot2 = a2[:H2], a2[H2:]

    h1, asad1 = _pre_call(xp, W1.T, b1.reshape(1, H1), _pack_a8(a1, H1))
    p = _make_sc_edge_kernel(H1, nchunk)(_bf16_interleave(h1), asad1,
                                         rowp, colp)
    h2, asad2 = _mid_call(p[0], p[1], h1, atop1, abot1, W2.T,
                          b2.reshape(1, H2), _pack_a8(a2, H2))
    q = _make_sc_edge_kernel(H2, nchunk)(_bf16_interleave(h2), asad2,
                                         rowp, colp)
    out = _post_call(q[0], q[1], h2, atop2, abot2)
    return out[:N]
